# baseline (speedup 1.0000x reference)
import dataclasses
import functools

import jax
import jax.numpy as jnp
from jax import lax
from jax.experimental import pallas as pl
from jax.experimental.pallas import tpu as pltpu
from jax.experimental.pallas import tpu_sc as plsc

_EPS = 1e-5
_LANES = 16
_CHUNK = 512
_IDXW = 128


@functools.cache
def _make_sc_kernel(BL: int, V: int, D: int):
    info = plsc.get_sparse_core_info()
    NC, NS = info.num_cores, info.num_subcores
    NW = NC * NS
    per_w = BL // NW
    n_chunks = per_w // _CHUNK
    n_gather = _CHUNK // _IDXW
    nv = D // _LANES
    mesh = plsc.VectorSubcoreMesh(core_axis_name="c", subcore_axis_name="s")
    cp = pltpu.CompilerParams()
    for fld, val in (("needs_layout_passes", False),
                     ("use_tc_tiling_on_sc", False)):
        if fld in pltpu.CompilerParams.__dataclass_fields__:
            cp = dataclasses.replace(cp, **{fld: val})

    @functools.partial(
        pl.kernel,
        compiler_params=cp,
        out_type=jax.ShapeDtypeStruct((BL, D), jnp.float32),
        mesh=mesh,
        scratch_types=[
            pltpu.VMEM((n_gather, _IDXW), jnp.int32),
            pltpu.VMEM((_CHUNK, D), jnp.float32),
            pltpu.VMEM((D,), jnp.float32),
            pltpu.VMEM((D,), jnp.float32),
            pltpu.SemaphoreType.DMA,
        ],
    )
    def k(idx_hbm, table_hbm, gamma_hbm, beta_hbm, out_hbm,
          idx_v, rows_v, g_v, b_v, sem):
        wid = lax.axis_index("s") * NC + lax.axis_index("c")
        pltpu.sync_copy(gamma_hbm, g_v)
        pltpu.sync_copy(beta_hbm, b_v)
        g = [g_v[pl.ds(i * _LANES, _LANES)] for i in range(nv)]
        b = [b_v[pl.ds(i * _LANES, _LANES)] for i in range(nv)]
        inv_d = jnp.float32(1.0 / D)

        @pl.loop(0, n_chunks)
        def _chunk(c):
            base = wid * per_w + c * _CHUNK
            irow = wid * (per_w // _IDXW) + c * n_gather
            pltpu.sync_copy(idx_hbm.at[pl.ds(irow, n_gather)], idx_v)
            cps = [
                pltpu.async_copy(
                    table_hbm.at[idx_v.at[j]],
                    rows_v.at[pl.ds(j * _IDXW, _IDXW)],
                    sem,
                )
                for j in range(n_gather)
            ]
            for cp in cps:
                cp.wait()

            @pl.loop(0, _CHUNK)
            def _row(r):
                v = [rows_v[r, pl.ds(i * _LANES, _LANES)] for i in range(nv)]
                s = v[0]
                q = v[0] * v[0]
                for i in range(1, nv):
                    s = s + v[i]
                    q = q + v[i] * v[i]
                mean = jnp.sum(s) * inv_d
                msq = jnp.sum(q) * inv_d
                var = msq - mean * mean
                x = jnp.broadcast_to(var + jnp.float32(_EPS), (_LANES,))
                i32 = lax.bitcast_convert_type(x, jnp.int32)
                i32 = jnp.int32(0x5F3759DF) - (i32 >> 1)
                y = lax.bitcast_convert_type(i32, jnp.float32)
                half = jnp.float32(-0.5) * x
                for _ in range(3):
                    y = y * (jnp.float32(1.5) + half * y * y)
                mv = jnp.broadcast_to(mean, (_LANES,))
                for i in range(nv):
                    rows_v[r, pl.ds(i * _LANES, _LANES)] = (
                        (v[i] - mv) * y * g[i] + b[i]
                    )

            pltpu.sync_copy(rows_v, out_hbm.at[pl.ds(base, _CHUNK)])

    return k


def kernel(input_ids, table, gamma, beta):
    B, L = input_ids.shape
    V, D = table.shape
    BL = B * L
    idx = input_ids.reshape(BL).astype(jnp.int32).reshape(BL // _IDXW, _IDXW)
    out_flat = _make_sc_kernel(BL, V, D)(idx, table, gamma, beta)
    return out_flat.reshape(B, L, D)


# speedup vs baseline: 1.5002x; 1.5002x over previous
import dataclasses
import functools

import jax
import jax.numpy as jnp
from jax import lax
from jax.experimental import pallas as pl
from jax.experimental.pallas import tpu as pltpu
from jax.experimental.pallas import tpu_sc as plsc

_EPS = 1e-5
_LANES = 16
_CHUNK = 256
_IDXW = 128
_NBUF = 4
_AHEAD = 2


@functools.cache
def _make_sc_kernel(BL: int, V: int, D: int):
    info = plsc.get_sparse_core_info()
    NC, NS = info.num_cores, info.num_subcores
    NW = NC * NS
    per_w = BL // NW
    n_chunks = per_w // _CHUNK
    n_gather = _CHUNK // _IDXW
    n_irows = per_w // _IDXW
    nv = D // _LANES
    mesh = plsc.VectorSubcoreMesh(core_axis_name="c", subcore_axis_name="s")
    cp = pltpu.CompilerParams()
    for fld, val in (("needs_layout_passes", False),
                     ("use_tc_tiling_on_sc", False)):
        if fld in pltpu.CompilerParams.__dataclass_fields__:
            cp = dataclasses.replace(cp, **{fld: val})

    @functools.partial(
        pl.kernel,
        compiler_params=cp,
        out_type=jax.ShapeDtypeStruct((BL, D), jnp.float32),
        mesh=mesh,
        scratch_types=[
            pltpu.VMEM((n_irows, _IDXW), jnp.int32),
            *[pltpu.VMEM((_CHUNK, D), jnp.float32) for _ in range(_NBUF)],
            pltpu.VMEM((D,), jnp.float32),
            pltpu.VMEM((D,), jnp.float32),
            *[pltpu.SemaphoreType.DMA for _ in range(2 * _NBUF)],
        ],
    )
    def k(idx_hbm, table_hbm, gamma_hbm, beta_hbm, out_hbm,
          idx_v, *rest):
        bufs = rest[:_NBUF]
        g_v, b_v = rest[_NBUF], rest[_NBUF + 1]
        semg = rest[_NBUF + 2:_NBUF + 2 + _NBUF]
        semo = rest[_NBUF + 2 + _NBUF:]
        wid = lax.axis_index("s") * NC + lax.axis_index("c")
        pltpu.sync_copy(idx_hbm.at[pl.ds(wid * n_irows, n_irows)], idx_v)
        pltpu.sync_copy(gamma_hbm, g_v)
        pltpu.sync_copy(beta_hbm, b_v)
        g = [g_v[pl.ds(i * _LANES, _LANES)] for i in range(nv)]
        b = [b_v[pl.ds(i * _LANES, _LANES)] for i in range(nv)]
        inv_d = jnp.float32(1.0 / D)
        rbase = wid * per_w

        def fire_g(c, buf, sem):
            for j in range(n_gather):
                pltpu.async_copy(
                    table_hbm.at[idx_v.at[c * n_gather + j]],
                    buf.at[pl.ds(j * _IDXW, _IDXW)],
                    sem,
                )

        def wait_g(buf, sem):
            pltpu.make_async_copy(table_hbm.at[pl.ds(0, _CHUNK)], buf, sem).wait()

        def fire_o(c, buf, sem):
            pltpu.async_copy(buf, out_hbm.at[pl.ds(rbase + c * _CHUNK, _CHUNK)], sem)

        def wait_o(buf, sem):
            pltpu.make_async_copy(buf, out_hbm.at[pl.ds(0, _CHUNK)], sem).wait()

        def compute(buf):
            @plsc.parallel_loop(0, _CHUNK, unroll=4)
            def _row(r):
                v = [buf[r, pl.ds(i * _LANES, _LANES)] for i in range(nv)]
                s = v[0]
                q = v[0] * v[0]
                for i in range(1, nv):
                    s = s + v[i]
                    q = q + v[i] * v[i]
                mean = jnp.sum(s) * inv_d
                var = jnp.sum(q) * inv_d - mean * mean
                x = var + jnp.float32(_EPS)
                seed = jnp.int32(0x5F3759DF) - (
                    lax.bitcast_convert_type(x, jnp.int32) >> 1)
                y = lax.bitcast_convert_type(seed, jnp.float32)
                nh = jnp.float32(-0.5) * x
                for _ in range(3):
                    y = y * (jnp.float32(1.5) + nh * y * y)
                mv = jnp.broadcast_to(mean, (_LANES,))
                yv = jnp.broadcast_to(y, (_LANES,))
                for i in range(nv):
                    buf[r, pl.ds(i * _LANES, _LANES)] = (
                        (v[i] - mv) * (yv * g[i]) + b[i]
                    )

        for c in range(_AHEAD):
            fire_g(c, bufs[c % _NBUF], semg[c % _NBUF])

        @pl.loop(0, n_chunks // _NBUF)
        def _t(t):
            c0 = t * _NBUF
            for bb in range(_NBUF):
                c = c0 + bb
                nb = (bb + _AHEAD) % _NBUF
                cn = c + _AHEAD

                @pl.when(cn < n_chunks)
                def _():
                    @pl.when(cn >= _NBUF)
                    def _():
                        wait_o(bufs[nb], semo[nb])
                    fire_g(cn, bufs[nb], semg[nb])

                wait_g(bufs[bb], semg[bb])
                compute(bufs[bb])
                fire_o(c, bufs[bb], semo[bb])

        for bb in range(_NBUF):
            wait_o(bufs[bb], semo[bb])

    return k


def kernel(input_ids, table, gamma, beta):
    B, L = input_ids.shape
    V, D = table.shape
    BL = B * L
    idx = input_ids.reshape(BL).astype(jnp.int32).reshape(BL // _IDXW, _IDXW)
    out_flat = _make_sc_kernel(BL, V, D)(idx, table, gamma, beta)
    return out_flat.reshape(B, L, D)


# speedup vs baseline: 1.5671x; 1.0446x over previous
import dataclasses
import functools

import jax
import jax.numpy as jnp
from jax import lax
from jax.experimental import pallas as pl
from jax.experimental.pallas import tpu as pltpu
from jax.experimental.pallas import tpu_sc as plsc

_EPS = 1e-5
_LANES = 16
_CB = 256
_IDXW = 128
_PITCH = 65


@functools.cache
def _make_sc_kernel(B: int, L: int, V: int, D: int):
    info = plsc.get_sparse_core_info()
    NC, NS = info.num_cores, info.num_subcores
    NW = NC * NS
    b_per_w = B // NW
    halves = b_per_w // _CB
    n_chunks = L * halves
    n_gather = _CB // _IDXW
    irows_per_l = B // _IDXW
    w_irows = b_per_w // _IDXW
    nv = D // _LANES
    nblk = _CB // _LANES
    mesh = plsc.VectorSubcoreMesh(core_axis_name="c", subcore_axis_name="s")
    cp = pltpu.CompilerParams()
    for fld, val in (("needs_layout_passes", False),
                     ("use_tc_tiling_on_sc", False)):
        if fld in pltpu.CompilerParams.__dataclass_fields__:
            cp = dataclasses.replace(cp, **{fld: val})

    @functools.partial(
        pl.kernel,
        compiler_params=cp,
        out_type=jax.ShapeDtypeStruct((L, D, B), jnp.float32),
        mesh=mesh,
        scratch_types=[
            pltpu.VMEM((L * w_irows, _IDXW), jnp.int32),
            *[pltpu.VMEM((_CB, D), jnp.float32) for _ in range(2)],
            *[pltpu.VMEM((_CB, _PITCH), jnp.float32) for _ in range(2)],
            *[pltpu.VMEM((D, _CB), jnp.float32) for _ in range(2)],
            pltpu.VMEM((D,), jnp.float32),
            pltpu.VMEM((D,), jnp.float32),
            *[pltpu.SemaphoreType.DMA for _ in range(4)],
        ],
    )
    def k(idx_hbm, table_hbm, gamma_hbm, beta_hbm, out_hbm, idx_v, *rest):
        bufa = rest[0:2]
        bufp = rest[2:4]
        tbuf = rest[4:6]
        g_v, b_v = rest[6], rest[7]
        semg = rest[8:10]
        semo = rest[10:12]
        wid = lax.axis_index("s") * NC + lax.axis_index("c")
        for l in range(L):
            pltpu.async_copy(
                idx_hbm.at[pl.ds(l * irows_per_l + wid * w_irows, w_irows)],
                idx_v.at[pl.ds(l * w_irows, w_irows)],
                semg[0],
            )
        pltpu.make_async_copy(
            idx_hbm.at[pl.ds(0, L * w_irows)], idx_v, semg[0]
        ).wait()
        pltpu.sync_copy(gamma_hbm, g_v)
        pltpu.sync_copy(beta_hbm, b_v)
        g = [g_v[pl.ds(i * _LANES, _LANES)] for i in range(nv)]
        b = [b_v[pl.ds(i * _LANES, _LANES)] for i in range(nv)]
        inv_d = jnp.float32(1.0 / D)
        b0w = wid * b_per_w

        def fire_g(l, h, st):
            for j in range(n_gather):
                pltpu.async_copy(
                    table_hbm.at[idx_v.at[l * w_irows + h * n_gather + j]],
                    bufa[st].at[pl.ds(j * _IDXW, _IDXW)],
                    semg[st],
                )

        def wait_g(st):
            pltpu.make_async_copy(
                table_hbm.at[pl.ds(0, _CB)], bufa[st], semg[st]
            ).wait()

        def fire_o(l, h, st):
            pltpu.async_copy(
                tbuf[st],
                out_hbm.at[l, :, pl.ds(b0w + h * _CB, _CB)],
                semo[st],
            )

        def wait_o(st):
            pltpu.make_async_copy(
                tbuf[st], out_hbm.at[0, :, pl.ds(0, _CB)], semo[st]
            ).wait()

        def compute(st):
            ba, bp, tb = bufa[st], bufp[st], tbuf[st]

            @plsc.parallel_loop(0, _CB, unroll=4)
            def _row(r):
                v = [ba[r, pl.ds(i * _LANES, _LANES)] for i in range(nv)]
                s = v[0]
                q = v[0] * v[0]
                for i in range(1, nv):
                    s = s + v[i]
                    q = q + v[i] * v[i]
                mean = jnp.sum(s) * inv_d
                var = jnp.sum(q) * inv_d - mean * mean
                x = var + jnp.float32(_EPS)
                seed = jnp.int32(0x5F3759DF) - (
                    lax.bitcast_convert_type(x, jnp.int32) >> 1)
                y = lax.bitcast_convert_type(seed, jnp.float32)
                nh = jnp.float32(-0.5) * x
                for _ in range(3):
                    y = y * (jnp.float32(1.5) + nh * y * y)
                mv = jnp.broadcast_to(mean, (_LANES,))
                yv = jnp.broadcast_to(y, (_LANES,))
                for i in range(nv):
                    bp[r, pl.ds(i * _LANES, _LANES)] = (
                        (v[i] - mv) * (yv * g[i]) + b[i]
                    )

            @plsc.parallel_loop(0, nblk)
            def _blk(blk):
                rows = blk * _LANES + jnp.arange(_LANES, dtype=jnp.int32)
                for j in range(D):
                    cols = jnp.full((_LANES,), j, jnp.int32)
                    xj = plsc.load_gather(bp, [rows, cols])
                    tb[j, pl.ds(blk * _LANES, _LANES)] = xj

        fire_g(0, 0, 0)

        @pl.loop(0, L)
        def _t(t):
            for st in range(2):
                if st == 0:
                    fire_g(t, 1, 1)
                else:
                    @pl.when(t < L - 1)
                    def _():
                        fire_g(t + 1, 0, 0)
                wait_g(st)

                @pl.when(t >= 1)
                def _():
                    wait_o(st)
                compute(st)
                fire_o(t, st, st)

        for st in range(2):
            wait_o(st)

    return k


def kernel(input_ids, table, gamma, beta):
    B, L = input_ids.shape
    V, D = table.shape
    idx = input_ids.T.astype(jnp.int32).reshape((B * L) // _IDXW, _IDXW)
    out3 = _make_sc_kernel(B, L, V, D)(idx, table, gamma, beta)
    return jnp.transpose(out3, (2, 0, 1))


# speedup vs baseline: 1.7419x; 1.1115x over previous
import dataclasses
import functools

import jax
import jax.numpy as jnp
from jax import lax
from jax.experimental import pallas as pl
from jax.experimental.pallas import tpu as pltpu
from jax.experimental.pallas import tpu_sc as plsc

_EPS = 1e-5
_LANES = 16
_CB = 256
_IDXW = 128
_PITCH = 65
_SL = 8


@functools.cache
def _make_sc_kernel(B: int, L: int, V: int, D: int):
    info = plsc.get_sparse_core_info()
    NC, NS = info.num_cores, info.num_subcores
    NW = NC * NS
    Lp = ((L + _SL - 1) // _SL) * _SL
    b_per_w = B // NW
    halves = b_per_w // _CB
    n_gather = _CB // _IDXW
    w_crows = b_per_w // _IDXW
    nv = D // _LANES
    nblk = _CB // _LANES
    cpchunk = _CB // _IDXW
    mesh = plsc.VectorSubcoreMesh(core_axis_name="c", subcore_axis_name="s")
    cp = pltpu.CompilerParams()
    for fld, val in (("needs_layout_passes", False),
                     ("use_tc_tiling_on_sc", False)):
        if fld in pltpu.CompilerParams.__dataclass_fields__:
            cp = dataclasses.replace(cp, **{fld: val})

    @functools.partial(
        pl.kernel,
        compiler_params=cp,
        out_type=jax.ShapeDtypeStruct((L, D // _SL, B // _IDXW, _SL, _IDXW),
                                      jnp.float32),
        mesh=mesh,
        scratch_types=[
            pltpu.VMEM((L * w_crows, _IDXW), jnp.int32),
            *[pltpu.VMEM((_CB, D), jnp.float32) for _ in range(2)],
            *[pltpu.VMEM((_CB, _PITCH), jnp.float32) for _ in range(2)],
            *[pltpu.VMEM((D // _SL, cpchunk, _SL, _IDXW), jnp.float32)
              for _ in range(2)],
            pltpu.VMEM((D,), jnp.float32),
            pltpu.VMEM((D,), jnp.float32),
            *[pltpu.SemaphoreType.DMA for _ in range(4)],
        ],
    )
    def k(idx_hbm, table_hbm, gamma_hbm, beta_hbm, out_hbm, idx_v, *rest):
        bufa = rest[0:2]
        bufp = rest[2:4]
        tbuf = rest[4:6]
        g_v, b_v = rest[6], rest[7]
        semg = rest[8:10]
        semo = rest[10:12]
        wid = lax.axis_index("s") * NC + lax.axis_index("c")
        for l in range(L):
            pltpu.async_copy(
                idx_hbm.at[l // _SL, pl.ds(wid * w_crows, w_crows), l % _SL],
                idx_v.at[pl.ds(l * w_crows, w_crows)],
                semg[0],
            )
        for l in range(L):
            pltpu.make_async_copy(
                idx_hbm.at[0, pl.ds(0, w_crows), 0],
                idx_v.at[pl.ds(0, w_crows)],
                semg[0],
            ).wait()
        pltpu.sync_copy(gamma_hbm, g_v)
        pltpu.sync_copy(beta_hbm, b_v)
        g = [g_v[pl.ds(i * _LANES, _LANES)] for i in range(nv)]
        b = [b_v[pl.ds(i * _LANES, _LANES)] for i in range(nv)]
        inv_d = jnp.float32(1.0 / D)

        def fire_g(l, h, st):
            for j in range(n_gather):
                pltpu.async_copy(
                    table_hbm.at[idx_v.at[l * w_crows + h * n_gather + j]],
                    bufa[st].at[pl.ds(j * _IDXW, _IDXW)],
                    semg[st],
                )

        def wait_g(st):
            pltpu.make_async_copy(
                table_hbm.at[pl.ds(0, _CB)], bufa[st], semg[st]
            ).wait()

        def fire_o(l, h, st):
            pltpu.async_copy(
                tbuf[st],
                out_hbm.at[l, :, pl.ds(wid * w_crows + h * cpchunk, cpchunk)],
                semo[st],
            )

        def wait_o(st):
            pltpu.make_async_copy(
                tbuf[st], out_hbm.at[0, :, pl.ds(0, cpchunk)], semo[st]
            ).wait()

        def compute(st):
            ba, bp, tb = bufa[st], bufp[st], tbuf[st]

            @plsc.parallel_loop(0, _CB, unroll=4)
            def _row(r):
                v = [ba[r, pl.ds(i * _LANES, _LANES)] for i in range(nv)]
                s = v[0]
                q = v[0] * v[0]
                for i in range(1, nv):
                    s = s + v[i]
                    q = q + v[i] * v[i]
                mean = jnp.sum(s) * inv_d
                var = jnp.sum(q) * inv_d - mean * mean
                x = var + jnp.float32(_EPS)
                seed = jnp.int32(0x5F3759DF) - (
                    lax.bitcast_convert_type(x, jnp.int32) >> 1)
                y = lax.bitcast_convert_type(seed, jnp.float32)
                nh = jnp.float32(-0.5) * x
                for _ in range(3):
                    y = y * (jnp.float32(1.5) + nh * y * y)
                mv = jnp.broadcast_to(mean, (_LANES,))
                yv = jnp.broadcast_to(y, (_LANES,))
                for i in range(nv):
                    bp[r, pl.ds(i * _LANES, _LANES)] = (
                        (v[i] - mv) * (yv * g[i]) + b[i]
                    )

            @plsc.parallel_loop(0, nblk)
            def _blk(blk):
                rows = blk * _LANES + jnp.arange(_LANES, dtype=jnp.int32)
                cb = blk // (_IDXW // _LANES)
                co = (blk % (_IDXW // _LANES)) * _LANES
                for j in range(D):
                    cols = jnp.full((_LANES,), j, jnp.int32)
                    xj = plsc.load_gather(bp, [rows, cols])
                    tb[j // _SL, cb, j % _SL, pl.ds(co, _LANES)] = xj

        fire_g(0, 0, 0)

        @pl.loop(0, L)
        def _t(t):
            for st in range(2):
                if st == 0:
                    fire_g(t, 1, 1)
                else:
                    @pl.when(t < L - 1)
                    def _():
                        fire_g(t + 1, 0, 0)
                wait_g(st)

                @pl.when(t >= 1)
                def _():
                    wait_o(st)
                compute(st)
                fire_o(t, st, st)

        for st in range(2):
            wait_o(st)

    return k


def kernel(input_ids, table, gamma, beta):
    B, L = input_ids.shape
    V, D = table.shape
    Lp = ((L + _SL - 1) // _SL) * _SL
    padded = jnp.pad(input_ids.astype(jnp.int32), ((0, 0), (0, Lp - L)))
    idx4 = (padded.T.reshape(Lp // _SL, _SL, B // _IDXW, _IDXW)
            .transpose(0, 2, 1, 3))
    out5 = _make_sc_kernel(B, L, V, D)(idx4, table, gamma, beta)
    out = (out5.transpose(0, 1, 3, 2, 4)
           .reshape(L, D, B)
           .transpose(2, 0, 1))
    return out
